# baseline (speedup 1.0000x reference)
import functools

import jax
import jax.numpy as jnp
from jax import lax
from jax.experimental import pallas as pl
from jax.experimental.pallas import tpu as pltpu
from jax.experimental.pallas import tpu_sc as plsc

_NC, _NS = 2, 16
_NW = _NC * _NS
_CH = 128


def _pb_body(tb_ref, wb_ref, out_ref):
    a = tb_ref[0]
    w = wb_ref[...]
    out_ref[0] = jnp.sum(a * w, axis=0, keepdims=True)


def _tc_pb(mat_bT, w_b):
    B, Eb, L1 = mat_bT.shape
    LB = 2048
    return pl.pallas_call(
        _pb_body,
        grid=(B, pl.cdiv(L1, LB)),
        in_specs=[
            pl.BlockSpec((1, Eb, LB), lambda b, i: (b, 0, i)),
            pl.BlockSpec((Eb, 1), lambda b, i: (0, 0)),
        ],
        out_specs=pl.BlockSpec((1, 1, LB), lambda b, i: (b, 0, i)),
        out_shape=jax.ShapeDtypeStruct((B, 1, L1), jnp.float32),
    )(mat_bT, w_b.reshape(Eb, 1))


def _sc_gather(pb_flat, flat_c, idx_b, idx_c, rows, ec):
    rows_w = rows // _NW
    n_ch = rows_w // _CH
    mesh = plsc.VectorSubcoreMesh(core_axis_name="c", subcore_axis_name="s")

    @functools.partial(
        pl.kernel,
        out_type=(jax.ShapeDtypeStruct((rows,), jnp.float32),
                  jax.ShapeDtypeStruct((rows, ec), jnp.float32)),
        mesh=mesh,
        scratch_types=[
            pltpu.VMEM((n_ch, _CH), jnp.int32),
            pltpu.VMEM((n_ch, _CH), jnp.int32),
            pltpu.VMEM((rows_w,), jnp.float32),
            pltpu.VMEM((rows_w, ec), jnp.float32),
            pltpu.SemaphoreType.DMA,
        ],
        compiler_params=pltpu.CompilerParams(use_tc_tiling_on_sc=False),
    )
    def gather_k(pb_h, c_h, ib_h, ic_h, pbg_h, cg_h, ib_v, ic_v, rb_v, rc_v, sem):
        wid = lax.axis_index("s") * _NC + lax.axis_index("c")
        base = wid * rows_w
        pltpu.sync_copy(ib_h.at[wid], ib_v)
        pltpu.sync_copy(ic_h.at[wid], ic_v)
        copies = []
        for j in range(n_ch):
            copies.append(pltpu.async_copy(
                pb_h.at[ib_v.at[j]], rb_v.at[pl.ds(j * _CH, _CH)], sem))
            copies.append(pltpu.async_copy(
                c_h.at[ic_v.at[j]], rc_v.at[pl.ds(j * _CH, _CH)], sem))
        for cp in copies:
            cp.wait()
        pltpu.sync_copy(rb_v, pbg_h.at[pl.ds(base, rows_w)])
        pltpu.sync_copy(rc_v, cg_h.at[pl.ds(base, rows_w)])

    return gather_k(pb_flat, flat_c, idx_b, idx_c)


def _tc_body(pbg_ref, cg_ref, dt_ref, wc_ref, l1b_ref, a_ref,
             bias_ref, l2_ref, l2b_ref, out_ref):
    cg = cg_ref[...]
    dt = dt_ref[...]
    rc = jnp.dot(cg, wc_ref[...], preferred_element_type=jnp.float32)
    x = jnp.dot(cg, l1b_ref[...], preferred_element_type=jnp.float32)
    x = x + dt * a_ref[...] + bias_ref[...]
    xa = 1.0 / (1.0 + jnp.exp(-x))
    t = jnp.dot(xa, l2_ref[...], preferred_element_type=jnp.float32)
    rate = pbg_ref[...] + rc + t + l2b_ref[...]
    out_ref[...] = jnp.maximum(rate, 0.0) + jnp.log1p(jnp.exp(-jnp.abs(rate)))


def kernel(mat_b, mat_c, arr_b_idx, arr_c_idx, arr_delta_t,
           w_b, w_c, lin1a_w, lin1a_b, lin1b_w, lin1b_b, lin2_w, lin2_b):
    B, L1, Eb = mat_b.shape
    _, L2, Ec = mat_c.shape
    L = arr_b_idx.shape[1]
    R = B * L
    rows_w = R // _NW
    n_ch = rows_w // _CH

    ib = arr_b_idx.reshape(-1).astype(jnp.int32)
    ic = arr_c_idx.reshape(-1).astype(jnp.int32)
    boff = jnp.arange(B, dtype=jnp.int32)[:, None]
    gb = (boff * L1 + ib[None, :]).reshape(_NW, n_ch, _CH)
    gc = (boff * L2 + ic[None, :]).reshape(_NW, n_ch, _CH)

    pb = _tc_pb(jnp.transpose(mat_b, (0, 2, 1)), w_b)

    pbg, cg = _sc_gather(pb.reshape(B * L1), mat_c.reshape(B * L2, Ec),
                         gb, gc, R, Ec)

    dt_col = arr_delta_t.astype(jnp.float32).reshape(R, 1)
    wc_col = w_c.reshape(Ec, 1)
    l1bT = lin1b_w.T
    a_row = lin1a_w.reshape(1, -1)
    bias_row = (lin1a_b + lin1b_b).reshape(1, -1)
    l2_col = lin2_w.reshape(-1, 1)
    l2b = lin2_b.reshape(1, 1)

    RB = 4096
    grid = R // RB
    H = lin1b_w.shape[0]
    out = pl.pallas_call(
        _tc_body,
        grid=(grid,),
        in_specs=[
            pl.BlockSpec((RB, 1), lambda i: (i, 0)),
            pl.BlockSpec((RB, Ec), lambda i: (i, 0)),
            pl.BlockSpec((RB, 1), lambda i: (i, 0)),
            pl.BlockSpec((Ec, 1), lambda i: (0, 0)),
            pl.BlockSpec((Ec, H), lambda i: (0, 0)),
            pl.BlockSpec((1, H), lambda i: (0, 0)),
            pl.BlockSpec((1, H), lambda i: (0, 0)),
            pl.BlockSpec((H, 1), lambda i: (0, 0)),
            pl.BlockSpec((1, 1), lambda i: (0, 0)),
        ],
        out_specs=pl.BlockSpec((RB, 1), lambda i: (i, 0)),
        out_shape=jax.ShapeDtypeStruct((R, 1), jnp.float32),
    )(pbg.reshape(R, 1), cg, dt_col, wc_col, l1bT, a_row, bias_row, l2_col, l2b)

    return out.reshape(B, L)


# speedup vs baseline: 1.2851x; 1.2851x over previous
import functools

import jax
import jax.numpy as jnp
from jax import lax
from jax.experimental import pallas as pl
from jax.experimental.pallas import tpu as pltpu
from jax.experimental.pallas import tpu_sc as plsc

_NC, _NS = 2, 16
_NW = _NC * _NS
_CH = 128


def _pb_body(tb_ref, wb_ref, out_ref):
    a = tb_ref[0]
    w = wb_ref[...]
    out_ref[0] = jnp.sum(a * w, axis=0, keepdims=True)


def _tc_pb(mat_bT, w_b):
    B, Eb, L1 = mat_bT.shape
    LB = 8192
    return pl.pallas_call(
        _pb_body,
        grid=(B, pl.cdiv(L1, LB)),
        in_specs=[
            pl.BlockSpec((1, Eb, LB), lambda b, i: (b, 0, i)),
            pl.BlockSpec((Eb, 1), lambda b, i: (0, 0)),
        ],
        out_specs=pl.BlockSpec((1, 1, LB), lambda b, i: (b, 0, i)),
        out_shape=jax.ShapeDtypeStruct((B, 1, L1), jnp.float32),
    )(mat_bT, w_b.reshape(Eb, 1))


def _sc_gather(pb_flat, flat_c, idx_b, idx_c, rows, ec):
    rows_w = rows // _NW
    n_ch = rows_w // _CH
    mesh = plsc.VectorSubcoreMesh(core_axis_name="c", subcore_axis_name="s")

    @functools.partial(
        pl.kernel,
        out_type=(jax.ShapeDtypeStruct((rows,), jnp.float32),
                  jax.ShapeDtypeStruct((rows, ec), jnp.float32)),
        mesh=mesh,
        scratch_types=[
            pltpu.VMEM((n_ch, _CH), jnp.int32),
            pltpu.VMEM((n_ch, _CH), jnp.int32),
            pltpu.VMEM((rows_w,), jnp.float32),
            pltpu.VMEM((rows_w, ec), jnp.float32),
            pltpu.SemaphoreType.DMA,
        ],
        compiler_params=pltpu.CompilerParams(use_tc_tiling_on_sc=False),
    )
    def gather_k(pb_h, c_h, ib_h, ic_h, pbg_h, cg_h, ib_v, ic_v, rb_v, rc_v, sem):
        wid = lax.axis_index("s") * _NC + lax.axis_index("c")
        base = wid * rows_w
        pltpu.sync_copy(ib_h.at[wid], ib_v)
        pltpu.sync_copy(ic_h.at[wid], ic_v)
        copies = []
        for j in range(n_ch):
            copies.append(pltpu.async_copy(
                pb_h.at[ib_v.at[j]], rb_v.at[pl.ds(j * _CH, _CH)], sem))
            copies.append(pltpu.async_copy(
                c_h.at[ic_v.at[j]], rc_v.at[pl.ds(j * _CH, _CH)], sem))
        for cp in copies:
            cp.wait()
        pltpu.sync_copy(rb_v, pbg_h.at[pl.ds(base, rows_w)])
        pltpu.sync_copy(rc_v, cg_h.at[pl.ds(base, rows_w)])

    return gather_k(pb_flat, flat_c, idx_b, idx_c)


def _tc_body(pbg_ref, cg_ref, dt_ref, wc_ref, l1b_ref, a_ref,
             bias_ref, l2_ref, l2b_ref, out_ref):
    cg = cg_ref[...]
    dt = dt_ref[...]
    rc = jnp.dot(cg, wc_ref[...], preferred_element_type=jnp.float32)
    x = jnp.dot(cg, l1b_ref[...], preferred_element_type=jnp.float32)
    x = x + dt * a_ref[...] + bias_ref[...]
    xa = 1.0 / (1.0 + jnp.exp(-x))
    t = jnp.dot(xa, l2_ref[...], preferred_element_type=jnp.float32)
    rate = pbg_ref[...] + rc + t + l2b_ref[...]
    out_ref[...] = jnp.maximum(rate, 0.0) + jnp.log1p(jnp.exp(-jnp.abs(rate)))


def kernel(mat_b, mat_c, arr_b_idx, arr_c_idx, arr_delta_t,
           w_b, w_c, lin1a_w, lin1a_b, lin1b_w, lin1b_b, lin2_w, lin2_b):
    B, L1, Eb = mat_b.shape
    _, L2, Ec = mat_c.shape
    L = arr_b_idx.shape[1]
    R = B * L
    rows_w = R // _NW
    n_ch = rows_w // _CH

    ib = arr_b_idx.reshape(-1).astype(jnp.int32)
    ic = arr_c_idx.reshape(-1).astype(jnp.int32)
    boff = jnp.arange(B, dtype=jnp.int32)[:, None]
    gb = (boff * L1 + ib[None, :]).reshape(_NW, n_ch, _CH)
    gc = (boff * L2 + ic[None, :]).reshape(_NW, n_ch, _CH)

    pb = _tc_pb(jnp.transpose(mat_b, (0, 2, 1)), w_b)

    pbg, cg = _sc_gather(pb.reshape(B * L1), mat_c.reshape(B * L2, Ec),
                         gb, gc, R, Ec)

    dt_col = arr_delta_t.astype(jnp.float32).reshape(R, 1)
    wc_col = w_c.reshape(Ec, 1)
    l1bT = lin1b_w.T
    a_row = lin1a_w.reshape(1, -1)
    bias_row = (lin1a_b + lin1b_b).reshape(1, -1)
    l2_col = lin2_w.reshape(-1, 1)
    l2b = lin2_b.reshape(1, 1)

    RB = 4096
    grid = R // RB
    H = lin1b_w.shape[0]
    out = pl.pallas_call(
        _tc_body,
        grid=(grid,),
        in_specs=[
            pl.BlockSpec((RB, 1), lambda i: (i, 0)),
            pl.BlockSpec((RB, Ec), lambda i: (i, 0)),
            pl.BlockSpec((RB, 1), lambda i: (i, 0)),
            pl.BlockSpec((Ec, 1), lambda i: (0, 0)),
            pl.BlockSpec((Ec, H), lambda i: (0, 0)),
            pl.BlockSpec((1, H), lambda i: (0, 0)),
            pl.BlockSpec((1, H), lambda i: (0, 0)),
            pl.BlockSpec((H, 1), lambda i: (0, 0)),
            pl.BlockSpec((1, 1), lambda i: (0, 0)),
        ],
        out_specs=pl.BlockSpec((RB, 1), lambda i: (i, 0)),
        out_shape=jax.ShapeDtypeStruct((R, 1), jnp.float32),
    )(pbg.reshape(R, 1), cg, dt_col, wc_col, l1bT, a_row, bias_row, l2_col, l2b)

    return out.reshape(B, L)


# speedup vs baseline: 1.3452x; 1.0467x over previous
import functools

import jax
import jax.numpy as jnp
from jax import lax
from jax.experimental import pallas as pl
from jax.experimental.pallas import tpu as pltpu
from jax.experimental.pallas import tpu_sc as plsc

_NC, _NS = 2, 16
_NW = _NC * _NS
_CH = 128


def _pb_body(tb_ref, wb_ref, out_ref):
    a = tb_ref[0]
    w = wb_ref[...]
    out_ref[0] = jnp.dot(w, a, preferred_element_type=jnp.float32)


def _tc_pb(mat_bT, w_b):
    B, Eb, L1 = mat_bT.shape
    LB = 16384
    return pl.pallas_call(
        _pb_body,
        grid=(B, pl.cdiv(L1, LB)),
        in_specs=[
            pl.BlockSpec((1, Eb, LB), lambda b, i: (b, 0, i)),
            pl.BlockSpec((1, Eb), lambda b, i: (0, 0)),
        ],
        out_specs=pl.BlockSpec((1, 1, LB), lambda b, i: (b, 0, i)),
        out_shape=jax.ShapeDtypeStruct((B, 1, L1), jnp.float32),
    )(mat_bT, w_b.reshape(1, Eb))


def _sc_gather(pb_flat, flat_c, idx_b, idx_c, rows, ec):
    rows_w = rows // _NW
    n_ch = rows_w // _CH
    mesh = plsc.VectorSubcoreMesh(core_axis_name="c", subcore_axis_name="s")

    @functools.partial(
        pl.kernel,
        out_type=(jax.ShapeDtypeStruct((rows,), jnp.float32),
                  jax.ShapeDtypeStruct((rows, ec), jnp.float32)),
        mesh=mesh,
        scratch_types=[
            pltpu.VMEM((n_ch, _CH), jnp.int32),
            pltpu.VMEM((n_ch, _CH), jnp.int32),
            pltpu.VMEM((rows_w,), jnp.float32),
            pltpu.VMEM((rows_w, ec), jnp.float32),
            pltpu.SemaphoreType.DMA,
        ],
        compiler_params=pltpu.CompilerParams(use_tc_tiling_on_sc=False),
    )
    def gather_k(pb_h, c_h, ib_h, ic_h, pbg_h, cg_h, ib_v, ic_v, rb_v, rc_v, sem):
        wid = lax.axis_index("s") * _NC + lax.axis_index("c")
        base = wid * rows_w
        pltpu.sync_copy(ib_h.at[wid], ib_v)
        pltpu.sync_copy(ic_h.at[wid], ic_v)
        copies = []
        for j in range(n_ch):
            copies.append(pltpu.async_copy(
                pb_h.at[ib_v.at[j]], rb_v.at[pl.ds(j * _CH, _CH)], sem))
            copies.append(pltpu.async_copy(
                c_h.at[ic_v.at[j]], rc_v.at[pl.ds(j * _CH, _CH)], sem))
        for cp in copies:
            cp.wait()
        pltpu.sync_copy(rb_v, pbg_h.at[pl.ds(base, rows_w)])
        pltpu.sync_copy(rc_v, cg_h.at[pl.ds(base, rows_w)])

    return gather_k(pb_flat, flat_c, idx_b, idx_c)


def _tc_body(pbg_ref, cg_ref, dt_ref, wc_ref, l1b_ref, a_ref,
             bias_ref, l2_ref, l2b_ref, out_ref):
    cg = cg_ref[...]
    dt = dt_ref[...]
    rc = jnp.dot(cg, wc_ref[...], preferred_element_type=jnp.float32)
    x = jnp.dot(cg, l1b_ref[...], preferred_element_type=jnp.float32)
    x = x + dt * a_ref[...] + bias_ref[...]
    xa = 1.0 / (1.0 + jnp.exp(-x))
    t = jnp.dot(xa, l2_ref[...], preferred_element_type=jnp.float32)
    rate = pbg_ref[...] + rc + t + l2b_ref[...]
    out_ref[...] = jnp.maximum(rate, 0.0) + jnp.log1p(jnp.exp(-jnp.abs(rate)))


def kernel(mat_b, mat_c, arr_b_idx, arr_c_idx, arr_delta_t,
           w_b, w_c, lin1a_w, lin1a_b, lin1b_w, lin1b_b, lin2_w, lin2_b):
    B, L1, Eb = mat_b.shape
    _, L2, Ec = mat_c.shape
    L = arr_b_idx.shape[1]
    R = B * L
    rows_w = R // _NW
    n_ch = rows_w // _CH

    ib = arr_b_idx.reshape(-1).astype(jnp.int32)
    ic = arr_c_idx.reshape(-1).astype(jnp.int32)
    boff = jnp.arange(B, dtype=jnp.int32)[:, None]
    gb = (boff * L1 + ib[None, :]).reshape(_NW, n_ch, _CH)
    gc = (boff * L2 + ic[None, :]).reshape(_NW, n_ch, _CH)

    pb = _tc_pb(jnp.transpose(mat_b, (0, 2, 1)), w_b)

    pbg, cg = _sc_gather(pb.reshape(B * L1), mat_c.reshape(B * L2, Ec),
                         gb, gc, R, Ec)

    dt_col = arr_delta_t.astype(jnp.float32).reshape(R, 1)
    wc_col = w_c.reshape(Ec, 1)
    l1bT = lin1b_w.T
    a_row = lin1a_w.reshape(1, -1)
    bias_row = (lin1a_b + lin1b_b).reshape(1, -1)
    l2_col = lin2_w.reshape(-1, 1)
    l2b = lin2_b.reshape(1, 1)

    RB = 4096
    grid = R // RB
    H = lin1b_w.shape[0]
    out = pl.pallas_call(
        _tc_body,
        grid=(grid,),
        in_specs=[
            pl.BlockSpec((RB, 1), lambda i: (i, 0)),
            pl.BlockSpec((RB, Ec), lambda i: (i, 0)),
            pl.BlockSpec((RB, 1), lambda i: (i, 0)),
            pl.BlockSpec((Ec, 1), lambda i: (0, 0)),
            pl.BlockSpec((Ec, H), lambda i: (0, 0)),
            pl.BlockSpec((1, H), lambda i: (0, 0)),
            pl.BlockSpec((1, H), lambda i: (0, 0)),
            pl.BlockSpec((H, 1), lambda i: (0, 0)),
            pl.BlockSpec((1, 1), lambda i: (0, 0)),
        ],
        out_specs=pl.BlockSpec((RB, 1), lambda i: (i, 0)),
        out_shape=jax.ShapeDtypeStruct((R, 1), jnp.float32),
    )(pbg.reshape(R, 1), cg, dt_col, wc_col, l1bT, a_row, bias_row, l2_col, l2b)

    return out.reshape(B, L)
